# Initial kernel scaffold; baseline (speedup 1.0000x reference)
import jax
import jax.numpy as jnp
from jax.experimental import pallas as pl


def kernel(pt_coordinates):
    raise NotImplementedError("write your pallas kernel here")


# speedup vs baseline: 2.7058x; 2.7058x over previous
import functools
import jax
import jax.numpy as jnp
from jax import lax
from jax.experimental import pallas as pl
from jax.experimental.pallas import tpu as pltpu
from jax.experimental.pallas import tpu_sc as plsc

_B, _C, _N = 8, 3, 16384
_M = 512
_L = 16
_CHUNKS = _N // _L


def _lane_permute(v, perm):
    dnums = lax.GatherDimensionNumbers(
        offset_dims=(), collapsed_slice_dims=(0,), start_index_map=(0,))
    return lax.gather(v, perm[:, None], dnums, (1,),
                      mode=lax.GatherScatterMode.PROMISE_IN_BOUNDS)


def _splat_at(ref, pos):
    return lax.broadcast(ref[pl.ds(pos, _L)][0], (_L,))


def _fps_body(pts_hbm, out_hbm, x_v, y_v, z_v, dists_v, idx_v):
    cid = lax.axis_index("c")
    sid = lax.axis_index("s")
    wid = sid * 2 + cid

    @pl.when(wid < _B)
    def _body():
        base_off = wid * (_C * _N)
        pltpu.sync_copy(pts_hbm.at[pl.ds(base_off, _N)], x_v.at[pl.ds(0, _N)])
        pltpu.sync_copy(pts_hbm.at[pl.ds(base_off + _N, _N)],
                        y_v.at[pl.ds(0, _N)])
        pltpu.sync_copy(pts_hbm.at[pl.ds(base_off + 2 * _N, _N)],
                        z_v.at[pl.ds(0, _N)])

        lanes = lax.iota(jnp.int32, _L)
        zeros = jnp.zeros((_L,), jnp.int32)
        intmax = jnp.full((_L,), 2147483647, jnp.int32)
        inf16 = jnp.full((_L,), jnp.inf, jnp.float32)

        def _init(j, carry):
            dists_v[pl.ds(j * _L, _L)] = inf16
            return carry

        lax.fori_loop(0, _CHUNKS, _init, 0)

        qx = _splat_at(x_v, 0)
        qy = _splat_at(y_v, 0)
        qz = _splat_at(z_v, 0)

        def _outer(i, carry):
            qx, qy, qz, pending = carry

            def _chunk(j, st):
                runmax, runidx = st
                base = j * _L
                dx = x_v[pl.ds(base, _L)] - qx
                dy = y_v[pl.ds(base, _L)] - qy
                dz = z_v[pl.ds(base, _L)] - qz
                d = dx * dx + dy * dy
                d = d + dz * dz
                nd = jnp.minimum(dists_v[pl.ds(base, _L)], d)
                dists_v[pl.ds(base, _L)] = nd
                gt = nd > runmax
                runmax = jnp.where(gt, nd, runmax)
                runidx = jnp.where(gt, lanes + base, runidx)
                return runmax, runidx

            runmax, runidx = lax.fori_loop(
                0, _CHUNKS, _chunk,
                (jnp.full((_L,), -jnp.inf, jnp.float32), zeros))

            for shift in (8, 4, 2, 1):
                perm = (lanes + shift) & (_L - 1)
                bv = _lane_permute(runmax, perm)
                bi = _lane_permute(runidx, perm)
                take = (bv > runmax) | ((bv == runmax) & (bi < runidx))
                runmax = jnp.where(take, bv, runmax)
                runidx = jnp.where(take, bi, runidx)
            nxt_s = runidx[0]

            pending = jnp.where(lanes == (i & (_L - 1)), runidx, pending)

            @pl.when((i & (_L - 1)) == _L - 1)
            def _flush():
                idx_v[pl.ds(i - (_L - 1), _L)] = pending

            qx = _splat_at(x_v, nxt_s)
            qy = _splat_at(y_v, nxt_s)
            qz = _splat_at(z_v, nxt_s)
            return qx, qy, qz, pending

        lax.fori_loop(1, _M, _outer, (qx, qy, qz, zeros))
        pltpu.sync_copy(idx_v, out_hbm.at[wid])


@functools.partial(
    pl.kernel,
    mesh=plsc.VectorSubcoreMesh(core_axis_name="c", subcore_axis_name="s"),
    out_type=jax.ShapeDtypeStruct((_B, _M), jnp.int32),
    scratch_types=[
        pltpu.VMEM((_N + _L,), jnp.float32),
        pltpu.VMEM((_N + _L,), jnp.float32),
        pltpu.VMEM((_N + _L,), jnp.float32),
        pltpu.VMEM((_N,), jnp.float32),
        pltpu.VMEM((_M,), jnp.int32),
    ],
)
def _fps(pts_hbm, out_hbm, x_v, y_v, z_v, dists_v, idx_v):
    _fps_body(pts_hbm, out_hbm, x_v, y_v, z_v, dists_v, idx_v)


def kernel(pt_coordinates):
    return _fps(pt_coordinates.reshape(-1))
